# baseline (speedup 1.0000x reference)
import jax
import jax.numpy as jnp
from jax.experimental import pallas as pl
from jax.experimental.pallas import tpu as pltpu
from jax.experimental.pallas import tpu_sc as plsc

QB = 16
GATHER_WINDOW = 128
POS_PAD = 64


def _search_body(f0_ref, f1t_ref, gi0_ref, gi1_ref):
    b = pl.program_id(0)
    q = pl.program_id(1)
    f0 = f0_ref[0]
    f1t = f1t_ref[0]
    F, n = f1t.shape

    acc = jnp.abs(f0[:, 0:1] - f1t[0:1, :])
    for f in range(1, F):
        acc = acc + jnp.abs(f0[:, f:f + 1] - f1t[f:f + 1, :])

    d0 = jnp.min(acc, axis=-1, keepdims=True)
    jidx = jax.lax.broadcasted_iota(jnp.int32, acc.shape, 1)
    idx0 = jnp.min(jnp.where(acc == d0, jidx, n), axis=-1, keepdims=True)
    masked = jnp.where(jidx == idx0, jnp.float32(jnp.inf), acc)
    d1 = jnp.min(masked, axis=-1, keepdims=True)
    idx1 = jnp.min(jnp.where(masked == d1, jidx, n), axis=-1, keepdims=True)

    base = b * n
    gi0_ref[0, pl.ds(q * QB, QB), :] = idx0 + base
    gi1_ref[0, pl.ds(q * QB, QB), :] = idx1 + base


def _sc_gather(table, gi0, gi1):
    M, W = table.shape
    row_t = jax.ShapeDtypeStruct((M, W), table.dtype)

    @pl.kernel(
        out_type=[row_t, row_t],
        mesh=plsc.VectorSubcoreMesh(core_axis_name="core",
                                    subcore_axis_name="subcore"),
    )
    def sc_kernel(tab_hbm, i0_hbm, i1_hbm, g0_hbm, g1_hbm):
        def body(i0_vmem, i1_vmem, g0_vmem, g1_vmem):
            pltpu.sync_copy(tab_hbm.at[i0_vmem.at[0]], g0_vmem)
            pltpu.sync_copy(tab_hbm.at[i1_vmem.at[0]], g1_vmem)

        pltpu.emit_pipeline(
            body,
            grid=(M // GATHER_WINDOW,),
            in_specs=[
                pl.BlockSpec((1, GATHER_WINDOW), index_map=lambda i: (0, i)),
                pl.BlockSpec((1, GATHER_WINDOW), index_map=lambda i: (0, i)),
            ],
            out_specs=[
                pl.BlockSpec((GATHER_WINDOW, W), index_map=lambda i: (i, 0)),
                pl.BlockSpec((GATHER_WINDOW, W), index_map=lambda i: (i, 0)),
            ],
            core_axis_name=("core", "subcore"),
            dimension_semantics=(pltpu.PARALLEL,),
        )(i0_hbm, i1_hbm, g0_hbm, g1_hbm)

    return sc_kernel(table, gi0, gi1)


def _refine_body(f0_ref, g0_ref, g1_ref, gi0_ref, gi1_ref,
                 pos_ref, w_ref):
    F = f0_ref.shape[1]
    f0 = f0_ref[...]
    a0 = jnp.sum(jnp.abs(f0 - g0_ref[:, :F]), axis=-1, keepdims=True)
    a1 = jnp.sum(jnp.abs(f0 - g1_ref[:, :F]), axis=-1, keepdims=True)
    gi0 = gi0_ref[...]
    gi1 = gi1_ref[...]
    swap = (a1 < a0) | ((a1 == a0) & (gi1 < gi0))
    d0 = jnp.where(swap, a1, a0)
    d1 = jnp.where(swap, a0, a1)
    pos_ref[...] = jnp.where(swap, g1_ref[:, F:F + 2], g0_ref[:, F:F + 2])
    w_ref[...] = 1.0 / (d1 / d0 - 1.0)


def kernel(feat0, feat1, pos1):
    B, N, F = feat0.shape
    M = B * N
    f1t = jnp.swapaxes(feat1, 1, 2)

    gi0, gi1 = pl.pallas_call(
        _search_body,
        grid=(B, N // QB),
        in_specs=[
            pl.BlockSpec((1, QB, F), lambda b, q: (b, q, 0)),
            pl.BlockSpec((1, F, N), lambda b, q: (b, 0, 0)),
        ],
        out_specs=[
            pl.BlockSpec((1, N, 1), lambda b, q: (b, 0, 0)),
            pl.BlockSpec((1, N, 1), lambda b, q: (b, 0, 0)),
        ],
        out_shape=[
            jax.ShapeDtypeStruct((B, N, 1), jnp.int32),
            jax.ShapeDtypeStruct((B, N, 1), jnp.int32),
        ],
        compiler_params=pltpu.CompilerParams(
            dimension_semantics=("parallel", "arbitrary"),
        ),
    )(feat0, f1t)

    pos_pad = jnp.pad(pos1.reshape(M, 2), ((0, 0), (0, POS_PAD - 2)))
    table = jnp.concatenate([feat1.reshape(M, F), pos_pad], axis=1)

    g0, g1 = _sc_gather(table, gi0.reshape(1, M), gi1.reshape(1, M))

    out_pos, out_w = pl.pallas_call(
        _refine_body,
        out_shape=[
            jax.ShapeDtypeStruct((M, 2), jnp.float32),
            jax.ShapeDtypeStruct((M, 1), jnp.float32),
        ],
    )(feat0.reshape(M, F), g0, g1, gi0.reshape(M, 1), gi1.reshape(M, 1))

    return out_pos.reshape(B, N, 2), out_w.reshape(B, N)


# speedup vs baseline: 1.3634x; 1.3634x over previous
import jax
import jax.numpy as jnp
from jax.experimental import pallas as pl
from jax.experimental.pallas import tpu as pltpu
from jax.experimental.pallas import tpu_sc as plsc

QB = 32
GATHER_WINDOW = 128
POS_PAD = 64


def _search_body(f0_ref, f1t_ref, gi0_ref, gi1_ref):
    b = pl.program_id(0)
    q = pl.program_id(1)
    f0 = f0_ref[0]
    f1t = f1t_ref[0]
    F, n = f1t.shape

    acc_a = jnp.abs(f0[:, 0:1] - f1t[0:1, :])
    acc_b = jnp.abs(f0[:, 1:2] - f1t[1:2, :])
    for f in range(2, F, 2):
        acc_a = acc_a + jnp.abs(f0[:, f:f + 1] - f1t[f:f + 1, :])
        acc_b = acc_b + jnp.abs(f0[:, f + 1:f + 2] - f1t[f + 1:f + 2, :])
    acc = acc_a + acc_b

    d0 = jnp.min(acc, axis=-1, keepdims=True)
    jidx = jax.lax.broadcasted_iota(jnp.int32, acc.shape, 1)
    idx0 = jnp.min(jnp.where(acc == d0, jidx, n), axis=-1, keepdims=True)
    masked = jnp.where(jidx == idx0, jnp.float32(jnp.inf), acc)
    d1 = jnp.min(masked, axis=-1, keepdims=True)
    idx1 = jnp.min(jnp.where(masked == d1, jidx, n), axis=-1, keepdims=True)

    base = b * n
    gi0_ref[0, pl.ds(q * QB, QB), :] = idx0 + base
    gi1_ref[0, pl.ds(q * QB, QB), :] = idx1 + base


def _sc_gather(table, gi0, gi1):
    M, W = table.shape
    row_t = jax.ShapeDtypeStruct((M, W), table.dtype)

    @pl.kernel(
        out_type=[row_t, row_t],
        mesh=plsc.VectorSubcoreMesh(core_axis_name="core",
                                    subcore_axis_name="subcore"),
    )
    def sc_kernel(tab_hbm, i0_hbm, i1_hbm, g0_hbm, g1_hbm):
        def body(i0_vmem, i1_vmem, g0_vmem, g1_vmem):
            pltpu.sync_copy(tab_hbm.at[i0_vmem.at[0]], g0_vmem)
            pltpu.sync_copy(tab_hbm.at[i1_vmem.at[0]], g1_vmem)

        pltpu.emit_pipeline(
            body,
            grid=(M // GATHER_WINDOW,),
            in_specs=[
                pl.BlockSpec((1, GATHER_WINDOW), index_map=lambda i: (0, i)),
                pl.BlockSpec((1, GATHER_WINDOW), index_map=lambda i: (0, i)),
            ],
            out_specs=[
                pl.BlockSpec((GATHER_WINDOW, W), index_map=lambda i: (i, 0)),
                pl.BlockSpec((GATHER_WINDOW, W), index_map=lambda i: (i, 0)),
            ],
            core_axis_name=("core", "subcore"),
            dimension_semantics=(pltpu.PARALLEL,),
        )(i0_hbm, i1_hbm, g0_hbm, g1_hbm)

    return sc_kernel(table, gi0, gi1)


def _refine_body(f0_ref, g0_ref, g1_ref, gi0_ref, gi1_ref,
                 pos_ref, w_ref):
    F = f0_ref.shape[1]
    f0 = f0_ref[...]
    a0 = jnp.sum(jnp.abs(f0 - g0_ref[:, :F]), axis=-1, keepdims=True)
    a1 = jnp.sum(jnp.abs(f0 - g1_ref[:, :F]), axis=-1, keepdims=True)
    gi0 = gi0_ref[...]
    gi1 = gi1_ref[...]
    swap = (a1 < a0) | ((a1 == a0) & (gi1 < gi0))
    d0 = jnp.where(swap, a1, a0)
    d1 = jnp.where(swap, a0, a1)
    pos_ref[...] = jnp.where(swap, g1_ref[:, F:F + 2], g0_ref[:, F:F + 2])
    w_ref[...] = 1.0 / (d1 / d0 - 1.0)


def kernel(feat0, feat1, pos1):
    B, N, F = feat0.shape
    M = B * N
    f1t = jnp.swapaxes(feat1, 1, 2)

    gi0, gi1 = pl.pallas_call(
        _search_body,
        grid=(B, N // QB),
        in_specs=[
            pl.BlockSpec((1, QB, F), lambda b, q: (b, q, 0)),
            pl.BlockSpec((1, F, N), lambda b, q: (b, 0, 0)),
        ],
        out_specs=[
            pl.BlockSpec((1, N, 1), lambda b, q: (b, 0, 0)),
            pl.BlockSpec((1, N, 1), lambda b, q: (b, 0, 0)),
        ],
        out_shape=[
            jax.ShapeDtypeStruct((B, N, 1), jnp.int32),
            jax.ShapeDtypeStruct((B, N, 1), jnp.int32),
        ],
        compiler_params=pltpu.CompilerParams(
            dimension_semantics=("parallel", "arbitrary"),
        ),
    )(feat0, f1t)

    pos_pad = jnp.pad(pos1.reshape(M, 2), ((0, 0), (0, POS_PAD - 2)))
    table = jnp.concatenate([feat1.reshape(M, F), pos_pad], axis=1)

    g0, g1 = _sc_gather(table, gi0.reshape(1, M), gi1.reshape(1, M))

    out_pos, out_w = pl.pallas_call(
        _refine_body,
        out_shape=[
            jax.ShapeDtypeStruct((M, 2), jnp.float32),
            jax.ShapeDtypeStruct((M, 1), jnp.float32),
        ],
    )(feat0.reshape(M, F), g0, g1, gi0.reshape(M, 1), gi1.reshape(M, 1))

    return out_pos.reshape(B, N, 2), out_w.reshape(B, N)


# speedup vs baseline: 1.9081x; 1.3995x over previous
import jax
import jax.numpy as jnp
from jax.experimental import pallas as pl
from jax.experimental.pallas import tpu as pltpu
from jax.experimental.pallas import tpu_sc as plsc

QB = 512
NB = 8
GATHER_WINDOW = 128
TABW = 128


def _accumulate(f0c, f1t):
    F = f1t.shape[0]
    h = f1t.shape[1] // 2
    fla = f1t[:, :h]
    flb = f1t[:, h:]
    c0 = f0c[:, 0:1]
    c1 = f0c[:, 1:2]
    acc_a0 = jnp.abs(c0 - fla[0:1, :])
    acc_b0 = jnp.abs(c0 - flb[0:1, :])
    acc_a1 = jnp.abs(c1 - fla[1:2, :])
    acc_b1 = jnp.abs(c1 - flb[1:2, :])
    for f in range(2, F, 2):
        c0 = f0c[:, f:f + 1]
        c1 = f0c[:, f + 1:f + 2]
        acc_a0 = acc_a0 + jnp.abs(c0 - fla[f:f + 1, :])
        acc_b0 = acc_b0 + jnp.abs(c0 - flb[f:f + 1, :])
        acc_a1 = acc_a1 + jnp.abs(c1 - fla[f + 1:f + 2, :])
        acc_b1 = acc_b1 + jnp.abs(c1 - flb[f + 1:f + 2, :])
    return jnp.concatenate([acc_a0 + acc_a1, acc_b0 + acc_b1], axis=-1)


def _search_body(f0_ref, f1_ref, pos1_ref, gi0_ref, gi1_ref, tab_ref):
    b = pl.program_id(0)
    q = pl.program_id(1)
    f0 = f0_ref[0]
    f1 = f1_ref[0]
    f1t = jnp.swapaxes(f1, 0, 1)
    n = f1t.shape[1]
    SUB = QB // NB

    pad = jnp.zeros((n, TABW - f1.shape[1] - 2), jnp.float32)
    tab_ref[0] = jnp.concatenate([f1, pos1_ref[0], pad], axis=-1)

    accs = [_accumulate(f0[i * SUB:(i + 1) * SUB], f1t) for i in range(NB)]
    jidx = jax.lax.broadcasted_iota(jnp.int32, accs[0].shape, 1)
    d0s = [jnp.min(a, axis=-1, keepdims=True) for a in accs]
    idx0s = [jnp.min(jnp.where(a == d0, jidx, n), axis=-1, keepdims=True)
             for a, d0 in zip(accs, d0s)]
    maskeds = [jnp.where(jidx == i0, jnp.float32(jnp.inf), a)
               for a, i0 in zip(accs, idx0s)]
    d1s = [jnp.min(m, axis=-1, keepdims=True) for m in maskeds]
    idx1s = [jnp.min(jnp.where(m == d1, jidx, n), axis=-1, keepdims=True)
             for m, d1 in zip(maskeds, d1s)]

    base = b * n
    idx0 = jnp.concatenate(idx0s, axis=0)
    idx1 = jnp.concatenate(idx1s, axis=0)
    gi0_ref[0, pl.ds(q * QB, QB), :] = idx0 + base
    gi1_ref[0, pl.ds(q * QB, QB), :] = idx1 + base


def _sc_gather(table, gi0, gi1):
    M, W = table.shape
    row_t = jax.ShapeDtypeStruct((M, W), table.dtype)

    @pl.kernel(
        out_type=[row_t, row_t],
        mesh=plsc.VectorSubcoreMesh(core_axis_name="core",
                                    subcore_axis_name="subcore"),
    )
    def sc_kernel(tab_hbm, i0_hbm, i1_hbm, g0_hbm, g1_hbm):
        def body(i0_vmem, i1_vmem, g0_vmem, g1_vmem):
            pltpu.sync_copy(tab_hbm.at[i0_vmem.at[0]], g0_vmem)
            pltpu.sync_copy(tab_hbm.at[i1_vmem.at[0]], g1_vmem)

        pltpu.emit_pipeline(
            body,
            grid=(M // GATHER_WINDOW,),
            in_specs=[
                pl.BlockSpec((1, GATHER_WINDOW), index_map=lambda i: (0, i)),
                pl.BlockSpec((1, GATHER_WINDOW), index_map=lambda i: (0, i)),
            ],
            out_specs=[
                pl.BlockSpec((GATHER_WINDOW, W), index_map=lambda i: (i, 0)),
                pl.BlockSpec((GATHER_WINDOW, W), index_map=lambda i: (i, 0)),
            ],
            core_axis_name=("core", "subcore"),
            dimension_semantics=(pltpu.PARALLEL,),
        )(i0_hbm, i1_hbm, g0_hbm, g1_hbm)

    return sc_kernel(table, gi0, gi1)


def _refine_body(f0_ref, g0_ref, g1_ref, gi0_ref, gi1_ref,
                 pos_ref, w_ref):
    F = f0_ref.shape[1]
    f0 = f0_ref[...]
    a0 = jnp.sum(jnp.abs(f0 - g0_ref[:, :F]), axis=-1, keepdims=True)
    a1 = jnp.sum(jnp.abs(f0 - g1_ref[:, :F]), axis=-1, keepdims=True)
    gi0 = gi0_ref[...]
    gi1 = gi1_ref[...]
    swap = (a1 < a0) | ((a1 == a0) & (gi1 < gi0))
    d0 = jnp.where(swap, a1, a0)
    d1 = jnp.where(swap, a0, a1)
    pos_ref[...] = jnp.where(swap, g1_ref[:, F:F + 2], g0_ref[:, F:F + 2])
    w_ref[...] = 1.0 / (d1 / d0 - 1.0)


def kernel(feat0, feat1, pos1):
    B, N, F = feat0.shape
    M = B * N

    gi0, gi1, tab = pl.pallas_call(
        _search_body,
        grid=(B, N // QB),
        in_specs=[
            pl.BlockSpec((1, QB, F), lambda b, q: (b, q, 0)),
            pl.BlockSpec((1, N, F), lambda b, q: (b, 0, 0)),
            pl.BlockSpec((1, N, 2), lambda b, q: (b, 0, 0)),
        ],
        out_specs=[
            pl.BlockSpec((1, N, 1), lambda b, q: (b, 0, 0)),
            pl.BlockSpec((1, N, 1), lambda b, q: (b, 0, 0)),
            pl.BlockSpec((1, N, TABW), lambda b, q: (b, 0, 0)),
        ],
        out_shape=[
            jax.ShapeDtypeStruct((B, N, 1), jnp.int32),
            jax.ShapeDtypeStruct((B, N, 1), jnp.int32),
            jax.ShapeDtypeStruct((B, N, TABW), jnp.float32),
        ],
        compiler_params=pltpu.CompilerParams(
            dimension_semantics=("parallel", "arbitrary"),
        ),
    )(feat0, feat1, pos1)

    g0, g1 = _sc_gather(tab.reshape(M, TABW),
                        gi0.reshape(1, M), gi1.reshape(1, M))

    out_pos, out_w = pl.pallas_call(
        _refine_body,
        out_shape=[
            jax.ShapeDtypeStruct((M, 2), jnp.float32),
            jax.ShapeDtypeStruct((M, 1), jnp.float32),
        ],
    )(feat0.reshape(M, F), g0, g1, gi0.reshape(M, 1), gi1.reshape(M, 1))

    return out_pos.reshape(B, N, 2), out_w.reshape(B, N)
